# baseline (speedup 1.0000x reference)
import jax
import jax.numpy as jnp
from jax.experimental import pallas as pl
from jax.experimental.pallas import tpu as pltpu

N = 5000
D_IN = 12544
D_HID = 1024
BM = 1000
BK = 896
NM = N // BM
NK = D_IN // BK


def _body(x_ref, w1_ref, b1_ref, w2_ref, b2_ref, wh_ref, bh_ref,
          out_ref, acc_ref):
    k = pl.program_id(0)
    m = pl.program_id(1)
    rows = pl.ds(m * BM, BM)
    part = jnp.dot(x_ref[...], w1_ref[...],
                   preferred_element_type=jnp.float32)

    @pl.when(k == 0)
    def _():
        acc_ref[rows, :] = part

    @pl.when(k > 0)
    def _():
        acc_ref[rows, :] += part

    @pl.when(k == NK - 1)
    def _():
        h1 = jnp.maximum(acc_ref[rows, :] + b1_ref[...], 0.0)
        h2 = jnp.maximum(
            jnp.dot(h1, w2_ref[...], preferred_element_type=jnp.float32)
            + b2_ref[...], 0.0)
        out_ref[...] = (
            jnp.dot(h2, wh_ref[...], preferred_element_type=jnp.float32)
            + bh_ref[...])


def kernel(feature_vectors, W1, b1, W2, b2, Wc, bc, Wr, br):
    wh = jnp.concatenate([Wc, Wr], axis=1)
    bh = jnp.concatenate([bc, br])[None, :]
    b1r = b1[None, :]
    b2r = b2[None, :]
    n_heads = wh.shape[1]

    out = pl.pallas_call(
        _body,
        grid=(NK, NM),
        in_specs=[
            pl.BlockSpec((BM, BK), lambda k, m: (m, k)),
            pl.BlockSpec((BK, D_HID), lambda k, m: (k, 0)),
            pl.BlockSpec((1, D_HID), lambda k, m: (0, 0)),
            pl.BlockSpec((D_HID, D_HID), lambda k, m: (0, 0)),
            pl.BlockSpec((1, D_HID), lambda k, m: (0, 0)),
            pl.BlockSpec((D_HID, n_heads), lambda k, m: (0, 0)),
            pl.BlockSpec((1, n_heads), lambda k, m: (0, 0)),
        ],
        out_specs=pl.BlockSpec((BM, n_heads), lambda k, m: (m, 0)),
        out_shape=jax.ShapeDtypeStruct((N, n_heads), jnp.float32),
        scratch_shapes=[pltpu.VMEM((N, D_HID), jnp.float32)],
        compiler_params=pltpu.CompilerParams(
            vmem_limit_bytes=100 * 1024 * 1024),
    )(feature_vectors, W1, b1r, W2, b2r, wh, bh)

    return out[:, :4], out[:, 4:]


# speedup vs baseline: 1.2133x; 1.2133x over previous
import jax
import jax.numpy as jnp
from jax.experimental import pallas as pl
from jax.experimental.pallas import tpu as pltpu

N = 5000
D_IN = 12544
D_HID = 1024
BM = 1000
BK = 1792
NM = N // BM
NK = D_IN // BK


def _body(x_ref, w1_ref, b1_ref, w2_ref, b2_ref, wh_ref, bh_ref,
          out_ref, acc_ref):
    k = pl.program_id(0)
    m = pl.program_id(1)
    rows = pl.ds(m * BM, BM)
    part = jnp.dot(x_ref[...], w1_ref[...],
                   preferred_element_type=jnp.float32)

    @pl.when(k == 0)
    def _():
        acc_ref[rows, :] = part

    @pl.when(k > 0)
    def _():
        acc_ref[rows, :] += part

    @pl.when(k == NK - 1)
    def _():
        h1 = jnp.maximum(acc_ref[rows, :] + b1_ref[...], 0.0)
        h2 = jnp.maximum(
            jnp.dot(h1, w2_ref[...], preferred_element_type=jnp.float32)
            + b2_ref[...], 0.0)
        out_ref[...] = (
            jnp.dot(h2, wh_ref[...], preferred_element_type=jnp.float32)
            + bh_ref[...])


def kernel(feature_vectors, W1, b1, W2, b2, Wc, bc, Wr, br):
    wh = jnp.concatenate([Wc, Wr], axis=1)
    bh = jnp.concatenate([bc, br])[None, :]
    b1r = b1[None, :]
    b2r = b2[None, :]
    n_heads = wh.shape[1]

    out = pl.pallas_call(
        _body,
        grid=(NK, NM),
        in_specs=[
            pl.BlockSpec((BM, BK), lambda k, m: (m, k)),
            pl.BlockSpec((BK, D_HID), lambda k, m: (k, 0)),
            pl.BlockSpec((1, D_HID), lambda k, m: (0, 0)),
            pl.BlockSpec((D_HID, D_HID), lambda k, m: (0, 0)),
            pl.BlockSpec((1, D_HID), lambda k, m: (0, 0)),
            pl.BlockSpec((D_HID, n_heads), lambda k, m: (0, 0)),
            pl.BlockSpec((1, n_heads), lambda k, m: (0, 0)),
        ],
        out_specs=pl.BlockSpec((BM, n_heads), lambda k, m: (m, 0)),
        out_shape=jax.ShapeDtypeStruct((N, n_heads), jnp.float32),
        scratch_shapes=[pltpu.VMEM((N, D_HID), jnp.float32)],
        compiler_params=pltpu.CompilerParams(
            vmem_limit_bytes=100 * 1024 * 1024,
            dimension_semantics=("arbitrary", "parallel")),
    )(feature_vectors, W1, b1r, W2, b2r, wh, bh)

    return out[:, :4], out[:, 4:]
